# baseline (speedup 1.0000x reference)
import jax
import jax.numpy as jnp
from jax.experimental import pallas as pl
from jax.experimental.pallas import tpu as pltpu

_BM = 512
_K = 5


def _body_factory(P, N, D, nb):
    nchunk_side = P * nb
    nchunk = 2 * nchunk_side
    inv_n = 1.0 / N

    def body(mps_d_ref, mps_p_ref, h_d_ref, h_p_ref, wt_ref, b_ref, a_ref,
             wfct_ref, bfc_ref, att_ref, zd_ref, zp_ref,
             ring, fts_scr, e_scr, st_scr, zstage_d, zstage_p,
             ring_sems, zd_sem, zp_sem):

        def chunk_copy(c, slot):
            side_p = c >= nchunk_side
            il = c - jnp.where(side_p, nchunk_side, 0)
            pq = il // nb
            ii = il - pq * nb

            @pl.when(jnp.logical_not(side_p))
            def _d():
                pltpu.make_async_copy(
                    mps_d_ref.at[pq, pl.ds(ii * _BM, _BM), :],
                    ring.at[slot], ring_sems.at[slot]).start()

            @pl.when(side_p)
            def _p():
                pltpu.make_async_copy(
                    mps_p_ref.at[pq, pl.ds(ii * _BM, _BM), :],
                    ring.at[slot], ring_sems.at[slot]).start()

        def chunk_wait(c, slot):
            side_p = c >= nchunk_side
            il = c - jnp.where(side_p, nchunk_side, 0)
            pq = il // nb
            ii = il - pq * nb

            @pl.when(jnp.logical_not(side_p))
            def _d():
                pltpu.make_async_copy(
                    mps_d_ref.at[pq, pl.ds(ii * _BM, _BM), :],
                    ring.at[slot], ring_sems.at[slot]).wait()

            @pl.when(side_p)
            def _p():
                pltpu.make_async_copy(
                    mps_p_ref.at[pq, pl.ds(ii * _BM, _BM), :],
                    ring.at[slot], ring_sems.at[slot]).wait()

        def combine(side, zstage):
            att = att_ref[side]
            ls = [jnp.sum(st_scr[q] * att, keepdims=True) * inv_n
                  for q in range(P)]
            m = ls[0]
            for q in range(1, P):
                m = jnp.maximum(m, ls[q])
            ws = [jnp.exp(l - m) for l in ls]
            den = ws[0]
            for q in range(1, P):
                den = den + ws[q]
            for q in range(P):
                beta = ws[q] / den
                if q == 0:
                    zstage[...] = e_scr[q].astype(jnp.float32) * beta
                else:
                    zstage[...] = zstage[...] + e_scr[q].astype(jnp.float32) * beta

        for c0 in range(_K - 1):
            chunk_copy(jnp.int32(c0), c0)

        def step(c, _):
            slot = jax.lax.rem(c, _K)

            @pl.when(c + _K - 1 < nchunk)
            def _issue():
                chunk_copy(c + _K - 1, jax.lax.rem(c + _K - 1, _K))

            chunk_wait(c, slot)

            side_p = c >= nchunk_side
            il = c - jnp.where(side_p, nchunk_side, 0)
            pq = il // nb
            ii = il - pq * nb

            for s_static in range(2):
                for q in range(P):
                    @pl.when(c == s_static * nchunk_side + q)
                    def _fts(s_static=s_static, q=q):
                        h = (h_d_ref if s_static == 0 else h_p_ref)[...]
                        fts = jnp.dot(h.astype(jnp.bfloat16),
                                      wt_ref[s_static * P + q],
                                      preferred_element_type=jnp.float32)
                        fts_scr[q] = fts.astype(jnp.bfloat16)

            sp = jnp.where(side_p, P, 0) + pq
            adj = ring[slot].astype(jnp.bfloat16)
            acc = jnp.dot(adj, fts_scr[pq], preferred_element_type=jnp.float32)
            out = acc + b_ref[sp]
            out = jnp.where(out >= 0, out, a_ref[sp] * out)
            e_scr[pq, pl.ds(ii * _BM, _BM), :] = out.astype(jnp.bfloat16)
            sdx = jnp.where(side_p, 1, 0)
            pre = jnp.dot(out.astype(jnp.bfloat16), wfct_ref[sdx],
                          preferred_element_type=jnp.float32) + bfc_ref[sdx]
            col = jnp.sum(jnp.tanh(pre), axis=0, keepdims=True)

            @pl.when(ii == 0)
            def _init():
                st_scr[pq] = col

            @pl.when(ii > 0)
            def _acc():
                st_scr[pq] = st_scr[pq] + col

            @pl.when(c == nchunk_side - 1)
            def _zd():
                combine(0, zstage_d)
                pltpu.make_async_copy(zstage_d, zd_ref, zd_sem).start()

            @pl.when(c == nchunk - 1)
            def _zp():
                combine(1, zstage_p)
                pltpu.make_async_copy(zstage_p, zp_ref, zp_sem).start()

            return ()

        jax.lax.fori_loop(0, nchunk, step, ())
        pltpu.make_async_copy(zstage_d, zd_ref, zd_sem).wait()
        pltpu.make_async_copy(zstage_p, zp_ref, zp_sem).wait()

    return body


def kernel(h_d, h_p, mps_d, mps_p, W_dg, b_dg, a_dg, W_pt, b_pt, a_pt,
           Wfc_d, bfc_d, att_d, Wfc_p, bfc_p, att_p):
    P, N, _ = mps_d.shape
    D = h_d.shape[1]
    nb = N // _BM

    Wt = jnp.concatenate([jnp.transpose(W_dg, (0, 2, 1)),
                          jnp.transpose(W_pt, (0, 2, 1))]).astype(jnp.bfloat16)
    b2 = jnp.concatenate([b_dg, b_pt]).reshape(2 * P, 1, D)
    a2 = jnp.broadcast_to(
        jnp.concatenate([a_dg, a_pt]).reshape(2 * P, 1, 1), (2 * P, 1, D))
    wfct = jnp.stack([Wfc_d.T, Wfc_p.T]).astype(jnp.bfloat16)
    bfc3 = jnp.stack([bfc_d, bfc_p]).reshape(2, 1, D)
    att3 = jnp.stack([att_d, att_p])

    vm = pltpu.VMEM
    z_d, z_p = pl.pallas_call(
        _body_factory(P, N, D, nb),
        in_specs=[
            pl.BlockSpec(memory_space=pl.ANY),
            pl.BlockSpec(memory_space=pl.ANY),
            pl.BlockSpec(memory_space=vm),
            pl.BlockSpec(memory_space=vm),
            pl.BlockSpec(memory_space=vm),
            pl.BlockSpec(memory_space=vm),
            pl.BlockSpec(memory_space=vm),
            pl.BlockSpec(memory_space=vm),
            pl.BlockSpec(memory_space=vm),
            pl.BlockSpec(memory_space=vm),
        ],
        out_specs=[
            pl.BlockSpec(memory_space=pl.ANY),
            pl.BlockSpec(memory_space=pl.ANY),
        ],
        out_shape=[
            jax.ShapeDtypeStruct((N, D), jnp.float32),
            jax.ShapeDtypeStruct((N, D), jnp.float32),
        ],
        scratch_shapes=[
            pltpu.VMEM((_K, _BM, N), jnp.float32),
            pltpu.VMEM((P, N, D), jnp.bfloat16),
            pltpu.VMEM((P, N, D), jnp.bfloat16),
            pltpu.VMEM((P, 1, D), jnp.float32),
            pltpu.VMEM((N, D), jnp.float32),
            pltpu.VMEM((N, D), jnp.float32),
            pltpu.SemaphoreType.DMA((_K,)),
            pltpu.SemaphoreType.DMA,
            pltpu.SemaphoreType.DMA,
        ],
    )(mps_d, mps_p, h_d, h_p, Wt, b2, a2, wfct, bfc3, att3)
    return (z_d, z_p)


# speedup vs baseline: 1.0045x; 1.0045x over previous
import jax
import jax.numpy as jnp
from jax.experimental import pallas as pl
from jax.experimental.pallas import tpu as pltpu

_BM = 256
_K = 8


def _body_factory(P, N, D, nb):
    nchunk_side = P * nb
    nchunk = 2 * nchunk_side
    inv_n = 1.0 / N

    def body(mps_d_ref, mps_p_ref, h_d_ref, h_p_ref, wt_ref, b_ref, a_ref,
             wfct_ref, bfc_ref, att_ref, zd_ref, zp_ref,
             ring, fts_scr, e_scr, st_scr, zstage_d, zstage_p,
             ring_sems, zd_sem, zp_sem):

        def chunk_copy(c, slot):
            side_p = c >= nchunk_side
            il = c - jnp.where(side_p, nchunk_side, 0)
            pq = il // nb
            ii = il - pq * nb

            @pl.when(jnp.logical_not(side_p))
            def _d():
                pltpu.make_async_copy(
                    mps_d_ref.at[pq, pl.ds(ii * _BM, _BM), :],
                    ring.at[slot], ring_sems.at[slot]).start()

            @pl.when(side_p)
            def _p():
                pltpu.make_async_copy(
                    mps_p_ref.at[pq, pl.ds(ii * _BM, _BM), :],
                    ring.at[slot], ring_sems.at[slot]).start()

        def chunk_wait(c, slot):
            side_p = c >= nchunk_side
            il = c - jnp.where(side_p, nchunk_side, 0)
            pq = il // nb
            ii = il - pq * nb

            @pl.when(jnp.logical_not(side_p))
            def _d():
                pltpu.make_async_copy(
                    mps_d_ref.at[pq, pl.ds(ii * _BM, _BM), :],
                    ring.at[slot], ring_sems.at[slot]).wait()

            @pl.when(side_p)
            def _p():
                pltpu.make_async_copy(
                    mps_p_ref.at[pq, pl.ds(ii * _BM, _BM), :],
                    ring.at[slot], ring_sems.at[slot]).wait()

        def combine(side, zstage):
            att = att_ref[side]
            ls = [jnp.sum(st_scr[q] * att, keepdims=True) * inv_n
                  for q in range(P)]
            m = ls[0]
            for q in range(1, P):
                m = jnp.maximum(m, ls[q])
            ws = [jnp.exp(l - m) for l in ls]
            den = ws[0]
            for q in range(1, P):
                den = den + ws[q]
            for q in range(P):
                beta = ws[q] / den
                if q == 0:
                    zstage[...] = e_scr[q].astype(jnp.float32) * beta
                else:
                    zstage[...] = zstage[...] + e_scr[q].astype(jnp.float32) * beta

        for c0 in range(_K - 1):
            chunk_copy(jnp.int32(c0), c0)

        def step(c, _):
            slot = jax.lax.rem(c, _K)

            @pl.when(c + _K - 1 < nchunk)
            def _issue():
                chunk_copy(c + _K - 1, jax.lax.rem(c + _K - 1, _K))

            chunk_wait(c, slot)

            side_p = c >= nchunk_side
            il = c - jnp.where(side_p, nchunk_side, 0)
            pq = il // nb
            ii = il - pq * nb

            for s_static in range(2):
                for q in range(P):
                    @pl.when(c == s_static * nchunk_side + q)
                    def _fts(s_static=s_static, q=q):
                        h = (h_d_ref if s_static == 0 else h_p_ref)[...]
                        fts = jnp.dot(h.astype(jnp.bfloat16),
                                      wt_ref[s_static * P + q],
                                      preferred_element_type=jnp.float32)
                        fts_scr[q] = fts.astype(jnp.bfloat16)

            sp = jnp.where(side_p, P, 0) + pq
            adj = ring[slot].astype(jnp.bfloat16)
            acc = jnp.dot(adj, fts_scr[pq], preferred_element_type=jnp.float32)
            out = acc + b_ref[sp]
            out = jnp.where(out >= 0, out, a_ref[sp] * out)
            e_scr[pq, pl.ds(ii * _BM, _BM), :] = out.astype(jnp.bfloat16)
            sdx = jnp.where(side_p, 1, 0)
            pre = jnp.dot(out.astype(jnp.bfloat16), wfct_ref[sdx],
                          preferred_element_type=jnp.float32) + bfc_ref[sdx]
            col = jnp.sum(jnp.tanh(pre), axis=0, keepdims=True)

            @pl.when(ii == 0)
            def _init():
                st_scr[pq] = col

            @pl.when(ii > 0)
            def _acc():
                st_scr[pq] = st_scr[pq] + col

            @pl.when(c == nchunk_side - 1)
            def _zd():
                combine(0, zstage_d)
                pltpu.make_async_copy(zstage_d, zd_ref, zd_sem).start()

            @pl.when(c == nchunk - 1)
            def _zp():
                combine(1, zstage_p)
                pltpu.make_async_copy(zstage_p, zp_ref, zp_sem).start()

            return ()

        jax.lax.fori_loop(0, nchunk, step, ())
        pltpu.make_async_copy(zstage_d, zd_ref, zd_sem).wait()
        pltpu.make_async_copy(zstage_p, zp_ref, zp_sem).wait()

    return body


def kernel(h_d, h_p, mps_d, mps_p, W_dg, b_dg, a_dg, W_pt, b_pt, a_pt,
           Wfc_d, bfc_d, att_d, Wfc_p, bfc_p, att_p):
    P, N, _ = mps_d.shape
    D = h_d.shape[1]
    nb = N // _BM

    Wt = jnp.concatenate([jnp.transpose(W_dg, (0, 2, 1)),
                          jnp.transpose(W_pt, (0, 2, 1))]).astype(jnp.bfloat16)
    b2 = jnp.concatenate([b_dg, b_pt]).reshape(2 * P, 1, D)
    a2 = jnp.broadcast_to(
        jnp.concatenate([a_dg, a_pt]).reshape(2 * P, 1, 1), (2 * P, 1, D))
    wfct = jnp.stack([Wfc_d.T, Wfc_p.T]).astype(jnp.bfloat16)
    bfc3 = jnp.stack([bfc_d, bfc_p]).reshape(2, 1, D)
    att3 = jnp.stack([att_d, att_p])

    vm = pltpu.VMEM
    z_d, z_p = pl.pallas_call(
        _body_factory(P, N, D, nb),
        in_specs=[
            pl.BlockSpec(memory_space=pl.ANY),
            pl.BlockSpec(memory_space=pl.ANY),
            pl.BlockSpec(memory_space=vm),
            pl.BlockSpec(memory_space=vm),
            pl.BlockSpec(memory_space=vm),
            pl.BlockSpec(memory_space=vm),
            pl.BlockSpec(memory_space=vm),
            pl.BlockSpec(memory_space=vm),
            pl.BlockSpec(memory_space=vm),
            pl.BlockSpec(memory_space=vm),
        ],
        out_specs=[
            pl.BlockSpec(memory_space=pl.ANY),
            pl.BlockSpec(memory_space=pl.ANY),
        ],
        out_shape=[
            jax.ShapeDtypeStruct((N, D), jnp.float32),
            jax.ShapeDtypeStruct((N, D), jnp.float32),
        ],
        scratch_shapes=[
            pltpu.VMEM((_K, _BM, N), jnp.float32),
            pltpu.VMEM((P, N, D), jnp.bfloat16),
            pltpu.VMEM((P, N, D), jnp.bfloat16),
            pltpu.VMEM((P, 1, D), jnp.float32),
            pltpu.VMEM((N, D), jnp.float32),
            pltpu.VMEM((N, D), jnp.float32),
            pltpu.SemaphoreType.DMA((_K,)),
            pltpu.SemaphoreType.DMA,
            pltpu.SemaphoreType.DMA,
        ],
    )(mps_d, mps_p, h_d, h_p, Wt, b2, a2, wfct, bfc3, att3)
    return (z_d, z_p)


# speedup vs baseline: 1.0164x; 1.0119x over previous
import jax
import jax.numpy as jnp
from jax.experimental import pallas as pl
from jax.experimental.pallas import tpu as pltpu

_BM = 512
_K = 4


def _body_factory(P, N, D, nb):
    nchunk_side = P * nb
    nchunk = 2 * nchunk_side
    inv_n = 1.0 / N

    def body(mps_d_ref, mps_p_ref, h_d_ref, h_p_ref, wt_ref, b_ref, a_ref,
             wfct_ref, bfc_ref, att_ref, zd_ref, zp_ref,
             ring, fts_scr, e_scr, st_scr, zstage, h_scr,
             ring_sems, zd_sem, zp_sem, h_sems):

        def chunk_copy(c, slot):
            side_p = c >= nchunk_side
            il = c - jnp.where(side_p, nchunk_side, 0)
            pq = il // nb
            ii = il - pq * nb

            @pl.when(jnp.logical_not(side_p))
            def _d():
                pltpu.make_async_copy(
                    mps_d_ref.at[pq, pl.ds(ii * _BM, _BM), :],
                    ring.at[slot], ring_sems.at[slot]).start()

            @pl.when(side_p)
            def _p():
                pltpu.make_async_copy(
                    mps_p_ref.at[pq, pl.ds(ii * _BM, _BM), :],
                    ring.at[slot], ring_sems.at[slot]).start()

        def chunk_wait(c, slot):
            side_p = c >= nchunk_side
            il = c - jnp.where(side_p, nchunk_side, 0)
            pq = il // nb
            ii = il - pq * nb

            @pl.when(jnp.logical_not(side_p))
            def _d():
                pltpu.make_async_copy(
                    mps_d_ref.at[pq, pl.ds(ii * _BM, _BM), :],
                    ring.at[slot], ring_sems.at[slot]).wait()

            @pl.when(side_p)
            def _p():
                pltpu.make_async_copy(
                    mps_p_ref.at[pq, pl.ds(ii * _BM, _BM), :],
                    ring.at[slot], ring_sems.at[slot]).wait()

        def combine(side, zstage):
            att = att_ref[side]
            ls = [jnp.sum(st_scr[q] * att, keepdims=True) * inv_n
                  for q in range(P)]
            m = ls[0]
            for q in range(1, P):
                m = jnp.maximum(m, ls[q])
            ws = [jnp.exp(l - m) for l in ls]
            den = ws[0]
            for q in range(1, P):
                den = den + ws[q]
            for q in range(P):
                beta = ws[q] / den
                if q == 0:
                    zstage[...] = e_scr[q].astype(jnp.float32) * beta
                else:
                    zstage[...] = zstage[...] + e_scr[q].astype(jnp.float32) * beta

        for c0 in range(_K - 1):
            chunk_copy(jnp.int32(c0), c0)
        pltpu.make_async_copy(h_d_ref, h_scr.at[0], h_sems.at[0]).start()
        pltpu.make_async_copy(h_p_ref, h_scr.at[1], h_sems.at[1]).start()

        def step(c, _):
            slot = jax.lax.rem(c, _K)

            @pl.when(c + _K - 1 < nchunk)
            def _issue():
                chunk_copy(c + _K - 1, jax.lax.rem(c + _K - 1, _K))

            chunk_wait(c, slot)

            side_p = c >= nchunk_side
            il = c - jnp.where(side_p, nchunk_side, 0)
            pq = il // nb
            ii = il - pq * nb

            for s_static in range(2):
                for q in range(P):
                    @pl.when(c == s_static * nchunk_side + q)
                    def _fts(s_static=s_static, q=q):
                        if q == 0:
                            pltpu.make_async_copy(
                                h_d_ref if s_static == 0 else h_p_ref,
                                h_scr.at[s_static],
                                h_sems.at[s_static]).wait()
                        fts = jnp.dot(h_scr[s_static].astype(jnp.bfloat16),
                                      wt_ref[s_static * P + q],
                                      preferred_element_type=jnp.float32)
                        fts_scr[q] = fts.astype(jnp.bfloat16)

            sp = jnp.where(side_p, P, 0) + pq
            adj = ring[slot].astype(jnp.bfloat16)
            acc = jnp.dot(adj, fts_scr[pq], preferred_element_type=jnp.float32)
            out = acc + b_ref[sp]
            out = jnp.where(out >= 0, out, a_ref[sp] * out)
            e_scr[pq, pl.ds(ii * _BM, _BM), :] = out.astype(jnp.bfloat16)
            sdx = jnp.where(side_p, 1, 0)
            pre = jnp.dot(out.astype(jnp.bfloat16), wfct_ref[sdx],
                          preferred_element_type=jnp.float32) + bfc_ref[sdx]
            col = jnp.sum(jnp.tanh(pre), axis=0, keepdims=True)

            @pl.when(ii == 0)
            def _init():
                st_scr[pq] = col

            @pl.when(ii > 0)
            def _acc():
                st_scr[pq] = st_scr[pq] + col

            @pl.when(c == nchunk_side - 1)
            def _zd():
                combine(0, zstage)
                pltpu.make_async_copy(zstage, zd_ref, zd_sem).start()

            @pl.when(c == nchunk - 1)
            def _zp():
                pltpu.make_async_copy(zstage, zd_ref, zd_sem).wait()
                combine(1, zstage)
                pltpu.make_async_copy(zstage, zp_ref, zp_sem).start()

            return ()

        jax.lax.fori_loop(0, nchunk, step, ())
        pltpu.make_async_copy(zstage, zp_ref, zp_sem).wait()

    return body


def kernel(h_d, h_p, mps_d, mps_p, W_dg, b_dg, a_dg, W_pt, b_pt, a_pt,
           Wfc_d, bfc_d, att_d, Wfc_p, bfc_p, att_p):
    P, N, _ = mps_d.shape
    D = h_d.shape[1]
    nb = N // _BM

    Wt = jnp.concatenate([jnp.transpose(W_dg, (0, 2, 1)),
                          jnp.transpose(W_pt, (0, 2, 1))]).astype(jnp.bfloat16)
    b2 = jnp.concatenate([b_dg, b_pt]).reshape(2 * P, 1, D)
    a2 = jnp.broadcast_to(
        jnp.concatenate([a_dg, a_pt]).reshape(2 * P, 1, 1), (2 * P, 1, D))
    wfct = jnp.stack([Wfc_d.T, Wfc_p.T]).astype(jnp.bfloat16)
    bfc3 = jnp.stack([bfc_d, bfc_p]).reshape(2, 1, D)
    att3 = jnp.stack([att_d, att_p])

    vm = pltpu.VMEM
    z_d, z_p = pl.pallas_call(
        _body_factory(P, N, D, nb),
        in_specs=[
            pl.BlockSpec(memory_space=pl.ANY),
            pl.BlockSpec(memory_space=pl.ANY),
            pl.BlockSpec(memory_space=pl.ANY),
            pl.BlockSpec(memory_space=pl.ANY),
            pl.BlockSpec(memory_space=vm),
            pl.BlockSpec(memory_space=vm),
            pl.BlockSpec(memory_space=vm),
            pl.BlockSpec(memory_space=vm),
            pl.BlockSpec(memory_space=vm),
            pl.BlockSpec(memory_space=vm),
        ],
        out_specs=[
            pl.BlockSpec(memory_space=pl.ANY),
            pl.BlockSpec(memory_space=pl.ANY),
        ],
        out_shape=[
            jax.ShapeDtypeStruct((N, D), jnp.float32),
            jax.ShapeDtypeStruct((N, D), jnp.float32),
        ],
        scratch_shapes=[
            pltpu.VMEM((_K, _BM, N), jnp.float32),
            pltpu.VMEM((P, N, D), jnp.bfloat16),
            pltpu.VMEM((P, N, D), jnp.bfloat16),
            pltpu.VMEM((P, 1, D), jnp.float32),
            pltpu.VMEM((N, D), jnp.float32),
            pltpu.VMEM((2, N, D), jnp.float32),
            pltpu.SemaphoreType.DMA((_K,)),
            pltpu.SemaphoreType.DMA,
            pltpu.SemaphoreType.DMA,
            pltpu.SemaphoreType.DMA((2,)),
        ],
    )(mps_d, mps_p, h_d, h_p, Wt, b2, a2, wfct, bfc3, att3)
    return (z_d, z_p)
